# Initial kernel scaffold; baseline (speedup 1.0000x reference)
import jax
import jax.numpy as jnp
from jax.experimental import pallas as pl


def kernel(z, weight):
    raise NotImplementedError("write your pallas kernel here")


# speedup vs baseline: 1.0282x; 1.0282x over previous
import functools

import jax
import jax.numpy as jnp
from jax import lax
from jax.experimental import pallas as pl
from jax.experimental.pallas import tpu as pltpu
from jax.experimental.pallas import tpu_sc as plsc

M = 8192
K = 8192
D = 256
TM = 512
TK = 2048

NC = 2
NS = 16
NW = NC * NS


def _argmin_body(zsq_ref, wsq_ref, z_ref, w_ref, out_ref, minval, minidx):
    j = pl.program_id(1)

    @pl.when(j == 0)
    def _init():
        minval[...] = jnp.full_like(minval[...], jnp.inf)
        minidx[...] = jnp.zeros_like(minidx[...])

    dot = lax.dot_general(z_ref[...], w_ref[...], (((1,), (1,)), ((), ())),
                          preferred_element_type=jnp.float32)
    d = (zsq_ref[...] + wsq_ref[...]) - 2.0 * dot
    local_min = jnp.min(d, axis=1, keepdims=True)
    lane = lax.broadcasted_iota(jnp.int32, (TM, TK), 1)
    cand = jnp.where(d == local_min, lane, TK)
    local_arg = jnp.min(cand, axis=1, keepdims=True) + j * TK

    better = local_min < minval[...]
    minval[...] = jnp.where(better, local_min, minval[...])
    minidx[...] = jnp.where(better, local_arg, minidx[...])

    @pl.when(j == pl.num_programs(1) - 1)
    def _emit():
        out_ref[...] = minidx[...]


_argmin_call = pl.pallas_call(
    _argmin_body,
    grid=(M // TM, K // TK),
    in_specs=[
        pl.BlockSpec((TM, 1), lambda i, j: (i, 0)),
        pl.BlockSpec((1, TK), lambda i, j: (0, j)),
        pl.BlockSpec((TM, D), lambda i, j: (i, 0)),
        pl.BlockSpec((TK, D), lambda i, j: (j, 0)),
    ],
    out_specs=pl.BlockSpec((TM, 1), lambda i, j: (i, 0)),
    out_shape=jax.ShapeDtypeStruct((M, 1), jnp.int32),
    scratch_shapes=[
        pltpu.VMEM((TM, 1), jnp.float32),
        pltpu.VMEM((TM, 1), jnp.int32),
    ],
)


@functools.cache
def _make_gather():
    b_per_w = M // NW
    mesh = plsc.VectorSubcoreMesh(core_axis_name="c", subcore_axis_name="s")

    @functools.partial(
        pl.kernel,
        mesh=mesh,
        out_type=jax.ShapeDtypeStruct((M, D), jnp.float32),
        scratch_types=[
            pltpu.VMEM((b_per_w,), jnp.int32),
            pltpu.VMEM((b_per_w, D), jnp.float32),
            pltpu.SemaphoreType.DMA,
        ],
    )
    def gather_k(table_hbm, idx_hbm, out_hbm, idx_v, rows_v, sem):
        wid = lax.axis_index("s") * NC + lax.axis_index("c")
        base = wid * b_per_w
        pltpu.sync_copy(idx_hbm.at[pl.ds(base, b_per_w)], idx_v)
        pltpu.async_copy(table_hbm.at[idx_v], rows_v, sem).wait()
        pltpu.sync_copy(rows_v, out_hbm.at[pl.ds(base, b_per_w)])

    return gather_k


def kernel(z, weight):
    B, C, H, W = z.shape
    zt = jnp.transpose(z, (0, 2, 3, 1))
    z_flat = zt.reshape(-1, C)
    zsq = jnp.sum(z_flat ** 2, axis=1, keepdims=True)
    wsq = jnp.sum(weight ** 2, axis=1)

    idx = _argmin_call(zsq, wsq[None, :], z_flat, weight).reshape(-1)
    z_q = _make_gather()(weight, idx)

    z_out = zt.reshape(B, H * W, C)
    return (z_out, z_q.reshape(B, H * W, C), idx.reshape(B, H * W))


# speedup vs baseline: 1.1377x; 1.1065x over previous
import functools

import jax
import jax.numpy as jnp
from jax import lax
from jax.experimental import pallas as pl
from jax.experimental.pallas import tpu as pltpu
from jax.experimental.pallas import tpu_sc as plsc

M = 8192
K = 8192
D = 256
TM = 512
TK = 2048

NC = 2
NS = 16
NW = NC * NS


def _argmin_body(zsq_ref, wsq_ref, zm2_ref, w_ref, out_ref, accv, accg):
    j = pl.program_id(1)

    @pl.when(j == 0)
    def _init():
        accv[...] = jnp.full_like(accv[...], jnp.inf)
        accg[...] = jnp.zeros_like(accg[...])

    dot2 = lax.dot_general(zm2_ref[...], w_ref[...], (((1,), (1,)), ((), ())),
                           preferred_element_type=jnp.float32)
    d = (zsq_ref[...] + wsq_ref[...]) + dot2

    av = accv[...]
    ag = accg[...]
    for g in range(TK // 128):
        dg = d[:, g * 128:(g + 1) * 128]
        better = dg < av
        av = jnp.where(better, dg, av)
        ag = jnp.where(better, j * (TK // 128) + g, ag)
    accv[...] = av
    accg[...] = ag

    @pl.when(j == pl.num_programs(1) - 1)
    def _emit():
        gidx = accg[...] * 128 + lax.broadcasted_iota(jnp.int32, (TM, 128), 1)
        rowmin = jnp.min(av, axis=1, keepdims=True)
        cand = jnp.where(av == rowmin, gidx, K)
        out_ref[...] = jnp.min(cand, axis=1, keepdims=True)


_argmin_call = pl.pallas_call(
    _argmin_body,
    grid=(M // TM, K // TK),
    in_specs=[
        pl.BlockSpec((TM, 1), lambda i, j: (i, 0)),
        pl.BlockSpec((1, TK), lambda i, j: (0, j)),
        pl.BlockSpec((TM, D), lambda i, j: (i, 0)),
        pl.BlockSpec((TK, D), lambda i, j: (j, 0)),
    ],
    out_specs=pl.BlockSpec((TM, 1), lambda i, j: (i, 0)),
    out_shape=jax.ShapeDtypeStruct((M, 1), jnp.int32),
    scratch_shapes=[
        pltpu.VMEM((TM, 128), jnp.float32),
        pltpu.VMEM((TM, 128), jnp.int32),
    ],
)


@functools.cache
def _make_gather():
    b_per_w = M // NW
    mesh = plsc.VectorSubcoreMesh(core_axis_name="c", subcore_axis_name="s")

    @functools.partial(
        pl.kernel,
        mesh=mesh,
        out_type=jax.ShapeDtypeStruct((M, D), jnp.float32),
        scratch_types=[
            pltpu.VMEM((b_per_w,), jnp.int32),
            pltpu.VMEM((b_per_w, D), jnp.float32),
            pltpu.SemaphoreType.DMA,
        ],
    )
    def gather_k(table_hbm, idx_hbm, out_hbm, idx_v, rows_v, sem):
        wid = lax.axis_index("s") * NC + lax.axis_index("c")
        base = wid * b_per_w
        pltpu.sync_copy(idx_hbm.at[pl.ds(base, b_per_w)], idx_v)
        pltpu.async_copy(table_hbm.at[idx_v], rows_v, sem).wait()
        pltpu.sync_copy(rows_v, out_hbm.at[pl.ds(base, b_per_w)])

    return gather_k


def kernel(z, weight):
    B, C, H, W = z.shape
    zt = jnp.transpose(z, (0, 2, 3, 1))
    z_flat = zt.reshape(-1, C)
    zsq = jnp.sum(z_flat ** 2, axis=1, keepdims=True)
    wsq = jnp.sum(weight ** 2, axis=1)
    zm2 = z_flat * (-2.0)

    idx = _argmin_call(zsq, wsq[None, :], zm2, weight).reshape(-1)
    z_q = _make_gather()(weight, idx)

    z_out = zt.reshape(B, H * W, C)
    return (z_out, z_q.reshape(B, H * W, C), idx.reshape(B, H * W))


# speedup vs baseline: 1.1558x; 1.0159x over previous
import functools

import jax
import jax.numpy as jnp
from jax import lax
from jax.experimental import pallas as pl
from jax.experimental.pallas import tpu as pltpu
from jax.experimental.pallas import tpu_sc as plsc

M = 8192
K = 8192
D = 256
TM = 512
TK = 2048
NCHUNK = 1

NC = 2
NS = 16
NW = NC * NS


def _argmin_body(zsq_ref, wsq_ref, z_ref, w_ref, out_ref, accv, accg):
    j = pl.program_id(1)

    @pl.when(j == 0)
    def _init():
        accv[...] = jnp.full_like(accv[...], jnp.inf)
        accg[...] = jnp.zeros_like(accg[...])

    dot2 = lax.dot_general(z_ref[...] * -2.0, w_ref[...],
                           (((1,), (1,)), ((), ())),
                           preferred_element_type=jnp.float32)
    d = (zsq_ref[...] + wsq_ref[...]) + dot2

    av = accv[...]
    ag = accg[...]
    for g in range(TK // 128):
        dg = d[:, g * 128:(g + 1) * 128]
        better = dg < av
        av = jnp.where(better, dg, av)
        ag = jnp.where(better, j * (TK // 128) + g, ag)
    accv[...] = av
    accg[...] = ag

    @pl.when(j == pl.num_programs(1) - 1)
    def _emit():
        gidx = accg[...] * 128 + lax.broadcasted_iota(jnp.int32, (TM, 128), 1)
        rowmin = jnp.min(av, axis=1, keepdims=True)
        cand = jnp.where(av == rowmin, gidx, K)
        out_ref[...] = jnp.min(cand, axis=1, keepdims=True)


def _make_argmin(mc):
    return pl.pallas_call(
        _argmin_body,
        grid=(mc // TM, K // TK),
        in_specs=[
            pl.BlockSpec((TM, 1), lambda i, j: (i, 0)),
            pl.BlockSpec((1, TK), lambda i, j: (0, j)),
            pl.BlockSpec((TM, D), lambda i, j: (i, 0)),
            pl.BlockSpec((TK, D), lambda i, j: (j, 0)),
        ],
        out_specs=pl.BlockSpec((TM, 1), lambda i, j: (i, 0)),
        out_shape=jax.ShapeDtypeStruct((mc, 1), jnp.int32),
        scratch_shapes=[
            pltpu.VMEM((TM, 128), jnp.float32),
            pltpu.VMEM((TM, 128), jnp.int32),
        ],
    )


@functools.cache
def _make_gather(b):
    b_per_w = b // NW
    mesh = plsc.VectorSubcoreMesh(core_axis_name="c", subcore_axis_name="s")

    @functools.partial(
        pl.kernel,
        mesh=mesh,
        out_type=jax.ShapeDtypeStruct((b, D), jnp.float32),
        scratch_types=[
            pltpu.VMEM((b_per_w,), jnp.int32),
            pltpu.VMEM((b_per_w, D), jnp.float32),
            pltpu.SemaphoreType.DMA,
        ],
    )
    def gather_k(table_hbm, idx_hbm, out_hbm, idx_v, rows_v, sem):
        wid = lax.axis_index("s") * NC + lax.axis_index("c")
        base = wid * b_per_w
        pltpu.sync_copy(idx_hbm.at[pl.ds(base, b_per_w)], idx_v)
        pltpu.async_copy(table_hbm.at[idx_v], rows_v, sem).wait()
        pltpu.sync_copy(rows_v, out_hbm.at[pl.ds(base, b_per_w)])

    return gather_k


def kernel(z, weight):
    B, C, H, W = z.shape
    zt = jnp.transpose(z, (0, 2, 3, 1))
    z_flat = zt.reshape(-1, C)
    zsq = jnp.sum(z_flat ** 2, axis=1, keepdims=True)
    wsq = jnp.sum(weight ** 2, axis=1)[None, :]

    mc = M // NCHUNK
    argmin_call = _make_argmin(mc)
    gather_call = _make_gather(mc)
    idx_parts = []
    zq_parts = []
    for c in range(NCHUNK):
        sl = slice(c * mc, (c + 1) * mc)
        idx_c = argmin_call(zsq[sl], wsq, z_flat[sl], weight).reshape(-1)
        idx_parts.append(idx_c)
        zq_parts.append(gather_call(weight, idx_c))

    idx = idx_parts[0] if NCHUNK == 1 else jnp.concatenate(idx_parts)
    z_q = zq_parts[0] if NCHUNK == 1 else jnp.concatenate(zq_parts)

    z_out = zt.reshape(B, H * W, C)
    return (z_out, z_q.reshape(B, H * W, C), idx.reshape(B, H * W))


# speedup vs baseline: 1.3833x; 1.1969x over previous
import functools

import jax
import jax.numpy as jnp
from jax import lax
from jax.experimental import pallas as pl
from jax.experimental.pallas import tpu as pltpu
from jax.experimental.pallas import tpu_sc as plsc

M = 8192
K = 8192
D = 256
TM = 512
TK = 2048

NC = 2
NS = 16
NW = NC * NS


def _argmin_body(zsq_ref, wsq_ref, z_ref, w_ref, out_ref):
    zm2 = z_ref[...] * -2.0
    zsq = zsq_ref[...]
    av = jnp.full((TM, 128), jnp.inf, jnp.float32)
    ag = jnp.zeros((TM, 128), jnp.int32)

    for j in range(K // TK):
        dot2 = lax.dot_general(zm2, w_ref[j * TK:(j + 1) * TK, :],
                               (((1,), (1,)), ((), ())),
                               preferred_element_type=jnp.float32)
        for g in range(TK // 128):
            dg = (zsq + wsq_ref[:, j * TK + g * 128:j * TK + (g + 1) * 128]) \
                + dot2[:, g * 128:(g + 1) * 128]
            better = dg < av
            av = jnp.where(better, dg, av)
            ag = jnp.where(better, j * (TK // 128) + g, ag)

    gidx = ag * 128 + lax.broadcasted_iota(jnp.int32, (TM, 128), 1)
    rowmin = jnp.min(av, axis=1, keepdims=True)
    cand = jnp.where(av == rowmin, gidx, K)
    out_ref[...] = jnp.min(cand, axis=1, keepdims=True)


_argmin_call = pl.pallas_call(
    _argmin_body,
    grid=(M // TM,),
    in_specs=[
        pl.BlockSpec((TM, 1), lambda i: (i, 0)),
        pl.BlockSpec((1, K), lambda i: (0, 0)),
        pl.BlockSpec((TM, D), lambda i: (i, 0)),
        pl.BlockSpec((K, D), lambda i: (0, 0)),
    ],
    out_specs=pl.BlockSpec((TM, 1), lambda i: (i, 0)),
    out_shape=jax.ShapeDtypeStruct((M, 1), jnp.int32),
)


@functools.cache
def _make_gather(b):
    b_per_w = b // NW
    mesh = plsc.VectorSubcoreMesh(core_axis_name="c", subcore_axis_name="s")

    @functools.partial(
        pl.kernel,
        mesh=mesh,
        out_type=jax.ShapeDtypeStruct((b, D), jnp.float32),
        scratch_types=[
            pltpu.VMEM((b_per_w,), jnp.int32),
            pltpu.VMEM((b_per_w, D), jnp.float32),
            pltpu.SemaphoreType.DMA,
        ],
    )
    def gather_k(table_hbm, idx_hbm, out_hbm, idx_v, rows_v, sem):
        wid = lax.axis_index("s") * NC + lax.axis_index("c")
        base = wid * b_per_w
        pltpu.sync_copy(idx_hbm.at[pl.ds(base, b_per_w)], idx_v)
        pltpu.async_copy(table_hbm.at[idx_v], rows_v, sem).wait()
        pltpu.sync_copy(rows_v, out_hbm.at[pl.ds(base, b_per_w)])

    return gather_k


def kernel(z, weight):
    B, C, H, W = z.shape
    zt = jnp.transpose(z, (0, 2, 3, 1))
    z_flat = zt.reshape(-1, C)
    zsq = jnp.sum(z_flat ** 2, axis=1, keepdims=True)
    wsq = jnp.sum(weight ** 2, axis=1)[None, :]

    idx = _argmin_call(zsq, wsq, z_flat, weight).reshape(-1)
    z_q = _make_gather(M)(weight, idx)

    z_out = zt.reshape(B, H * W, C)
    return (z_out, z_q.reshape(B, H * W, C), idx.reshape(B, H * W))


# speedup vs baseline: 1.4119x; 1.0207x over previous
import functools

import jax
import jax.numpy as jnp
from jax import lax
from jax.experimental import pallas as pl
from jax.experimental.pallas import tpu as pltpu
from jax.experimental.pallas import tpu_sc as plsc

M = 8192
K = 8192
D = 256
TM = 1024
TK = 2048

NC = 2
NS = 16
NW = NC * NS


def _argmin_body(zsq_ref, wsq_ref, z_ref, w_ref, out_ref):
    zm2 = z_ref[...] * -2.0
    zsq = zsq_ref[...]
    av = jnp.full((TM, 128), jnp.inf, jnp.float32)
    ag = jnp.zeros((TM, 128), jnp.int32)

    for j in range(K // TK):
        dot2 = lax.dot_general(zm2, w_ref[j * TK:(j + 1) * TK, :],
                               (((1,), (1,)), ((), ())),
                               preferred_element_type=jnp.float32)
        for g in range(TK // 128):
            dg = (zsq + wsq_ref[:, j * TK + g * 128:j * TK + (g + 1) * 128]) \
                + dot2[:, g * 128:(g + 1) * 128]
            better = dg < av
            av = jnp.where(better, dg, av)
            ag = jnp.where(better, j * (TK // 128) + g, ag)

    gidx = ag * 128 + lax.broadcasted_iota(jnp.int32, (TM, 128), 1)
    rowmin = jnp.min(av, axis=1, keepdims=True)
    cand = jnp.where(av == rowmin, gidx, K)
    out_ref[...] = jnp.min(cand, axis=1, keepdims=True)


_argmin_call = pl.pallas_call(
    _argmin_body,
    grid=(M // TM,),
    in_specs=[
        pl.BlockSpec((TM, 1), lambda i: (i, 0)),
        pl.BlockSpec((1, K), lambda i: (0, 0)),
        pl.BlockSpec((TM, D), lambda i: (i, 0)),
        pl.BlockSpec((K, D), lambda i: (0, 0)),
    ],
    out_specs=pl.BlockSpec((TM, 1), lambda i: (i, 0)),
    out_shape=jax.ShapeDtypeStruct((M, 1), jnp.int32),
)


@functools.cache
def _make_gather(b):
    b_per_w = b // NW
    mesh = plsc.VectorSubcoreMesh(core_axis_name="c", subcore_axis_name="s")

    @functools.partial(
        pl.kernel,
        mesh=mesh,
        out_type=jax.ShapeDtypeStruct((b, D), jnp.float32),
        scratch_types=[
            pltpu.VMEM((b_per_w,), jnp.int32),
            pltpu.VMEM((b_per_w, D), jnp.float32),
            pltpu.SemaphoreType.DMA,
        ],
    )
    def gather_k(table_hbm, idx_hbm, out_hbm, idx_v, rows_v, sem):
        wid = lax.axis_index("s") * NC + lax.axis_index("c")
        base = wid * b_per_w
        pltpu.sync_copy(idx_hbm.at[pl.ds(base, b_per_w)], idx_v)
        pltpu.async_copy(table_hbm.at[idx_v], rows_v, sem).wait()
        pltpu.sync_copy(rows_v, out_hbm.at[pl.ds(base, b_per_w)])

    return gather_k


def kernel(z, weight):
    B, C, H, W = z.shape
    zt = jnp.transpose(z, (0, 2, 3, 1))
    z_flat = zt.reshape(-1, C)
    zsq = jnp.sum(z_flat ** 2, axis=1, keepdims=True)
    wsq = jnp.sum(weight ** 2, axis=1)[None, :]

    idx = _argmin_call(zsq, wsq, z_flat, weight).reshape(-1)
    z_q = _make_gather(M)(weight, idx)

    z_out = zt.reshape(B, H * W, C)
    return (z_out, z_q.reshape(B, H * W, C), idx.reshape(B, H * W))


# speedup vs baseline: 1.4532x; 1.0292x over previous
import functools

import jax
import jax.numpy as jnp
from jax import lax
from jax.experimental import pallas as pl
from jax.experimental.pallas import tpu as pltpu
from jax.experimental.pallas import tpu_sc as plsc

M = 8192
K = 8192
D = 256
TM = 2048
TK = 2048

NC = 2
NS = 16
NW = NC * NS


def _argmin_body(zsq_ref, wsq_ref, z_ref, w_ref, out_ref):
    zm2 = z_ref[...] * -2.0
    zsq = zsq_ref[...]
    av = jnp.full((TM, 128), jnp.inf, jnp.float32)
    ag = jnp.zeros((TM, 128), jnp.int32)

    for j in range(K // TK):
        dot2 = lax.dot_general(zm2, w_ref[j * TK:(j + 1) * TK, :],
                               (((1,), (1,)), ((), ())),
                               preferred_element_type=jnp.float32)
        for g in range(TK // 128):
            dg = (zsq + wsq_ref[:, j * TK + g * 128:j * TK + (g + 1) * 128]) \
                + dot2[:, g * 128:(g + 1) * 128]
            better = dg < av
            av = jnp.where(better, dg, av)
            ag = jnp.where(better, j * (TK // 128) + g, ag)

    gidx = ag * 128 + lax.broadcasted_iota(jnp.int32, (TM, 128), 1)
    rowmin = jnp.min(av, axis=1, keepdims=True)
    cand = jnp.where(av == rowmin, gidx, K)
    out_ref[...] = jnp.min(cand, axis=1, keepdims=True)


_argmin_call = pl.pallas_call(
    _argmin_body,
    grid=(M // TM,),
    in_specs=[
        pl.BlockSpec((TM, 1), lambda i: (i, 0)),
        pl.BlockSpec((1, K), lambda i: (0, 0)),
        pl.BlockSpec((TM, D), lambda i: (i, 0)),
        pl.BlockSpec((K, D), lambda i: (0, 0)),
    ],
    out_specs=pl.BlockSpec((TM, 1), lambda i: (i, 0)),
    out_shape=jax.ShapeDtypeStruct((M, 1), jnp.int32),
)


@functools.cache
def _make_gather(b):
    b_per_w = b // NW
    mesh = plsc.VectorSubcoreMesh(core_axis_name="c", subcore_axis_name="s")

    @functools.partial(
        pl.kernel,
        mesh=mesh,
        out_type=jax.ShapeDtypeStruct((b, D), jnp.float32),
        scratch_types=[
            pltpu.VMEM((b_per_w,), jnp.int32),
            pltpu.VMEM((b_per_w, D), jnp.float32),
            pltpu.SemaphoreType.DMA,
        ],
    )
    def gather_k(table_hbm, idx_hbm, out_hbm, idx_v, rows_v, sem):
        wid = lax.axis_index("s") * NC + lax.axis_index("c")
        base = wid * b_per_w
        pltpu.sync_copy(idx_hbm.at[pl.ds(base, b_per_w)], idx_v)
        pltpu.async_copy(table_hbm.at[idx_v], rows_v, sem).wait()
        pltpu.sync_copy(rows_v, out_hbm.at[pl.ds(base, b_per_w)])

    return gather_k


def kernel(z, weight):
    B, C, H, W = z.shape
    zt = jnp.transpose(z, (0, 2, 3, 1))
    z_flat = zt.reshape(-1, C)
    zsq = jnp.sum(z_flat ** 2, axis=1, keepdims=True)
    wsq = jnp.sum(weight ** 2, axis=1)[None, :]

    idx = _argmin_call(zsq, wsq, z_flat, weight).reshape(-1)
    z_q = _make_gather(M)(weight, idx)

    z_out = zt.reshape(B, H * W, C)
    return (z_out, z_q.reshape(B, H * W, C), idx.reshape(B, H * W))


# speedup vs baseline: 1.4603x; 1.0049x over previous
import functools

import jax
import jax.numpy as jnp
from jax import lax
from jax.experimental import pallas as pl
from jax.experimental.pallas import tpu as pltpu
from jax.experimental.pallas import tpu_sc as plsc

M = 8192
K = 8192
D = 256
TM = 4096
TK = 2048

NC = 2
NS = 16
NW = NC * NS


def _argmin_body(zsq_ref, wsq_ref, z_ref, w_ref, out_ref):
    zm2 = z_ref[...] * -2.0
    zsq = zsq_ref[...]
    av = jnp.full((TM, 128), jnp.inf, jnp.float32)
    ag = jnp.zeros((TM, 128), jnp.int32)

    for j in range(K // TK):
        dot2 = lax.dot_general(zm2, w_ref[j * TK:(j + 1) * TK, :],
                               (((1,), (1,)), ((), ())),
                               preferred_element_type=jnp.float32)
        for g in range(TK // 128):
            dg = (zsq + wsq_ref[:, j * TK + g * 128:j * TK + (g + 1) * 128]) \
                + dot2[:, g * 128:(g + 1) * 128]
            better = dg < av
            av = jnp.where(better, dg, av)
            ag = jnp.where(better, j * (TK // 128) + g, ag)

    gidx = ag * 128 + lax.broadcasted_iota(jnp.int32, (TM, 128), 1)
    rowmin = jnp.min(av, axis=1, keepdims=True)
    cand = jnp.where(av == rowmin, gidx, K)
    out_ref[...] = jnp.min(cand, axis=1, keepdims=True)


_argmin_call = pl.pallas_call(
    _argmin_body,
    grid=(M // TM,),
    in_specs=[
        pl.BlockSpec((TM, 1), lambda i: (i, 0)),
        pl.BlockSpec((1, K), lambda i: (0, 0)),
        pl.BlockSpec((TM, D), lambda i: (i, 0)),
        pl.BlockSpec((K, D), lambda i: (0, 0)),
    ],
    out_specs=pl.BlockSpec((TM, 1), lambda i: (i, 0)),
    out_shape=jax.ShapeDtypeStruct((M, 1), jnp.int32),
)


@functools.cache
def _make_gather(b):
    b_per_w = b // NW
    mesh = plsc.VectorSubcoreMesh(core_axis_name="c", subcore_axis_name="s")

    @functools.partial(
        pl.kernel,
        mesh=mesh,
        out_type=jax.ShapeDtypeStruct((b, D), jnp.float32),
        scratch_types=[
            pltpu.VMEM((b_per_w,), jnp.int32),
            pltpu.VMEM((b_per_w, D), jnp.float32),
            pltpu.SemaphoreType.DMA,
        ],
    )
    def gather_k(table_hbm, idx_hbm, out_hbm, idx_v, rows_v, sem):
        wid = lax.axis_index("s") * NC + lax.axis_index("c")
        base = wid * b_per_w
        pltpu.sync_copy(idx_hbm.at[pl.ds(base, b_per_w)], idx_v)
        pltpu.async_copy(table_hbm.at[idx_v], rows_v, sem).wait()
        pltpu.sync_copy(rows_v, out_hbm.at[pl.ds(base, b_per_w)])

    return gather_k


def kernel(z, weight):
    B, C, H, W = z.shape
    zt = jnp.transpose(z, (0, 2, 3, 1))
    z_flat = zt.reshape(-1, C)
    zsq = jnp.sum(z_flat ** 2, axis=1, keepdims=True)
    wsq = jnp.sum(weight ** 2, axis=1)[None, :]

    idx = _argmin_call(zsq, wsq, z_flat, weight).reshape(-1)
    z_q = _make_gather(M)(weight, idx)

    z_out = zt.reshape(B, H * W, C)
    return (z_out, z_q.reshape(B, H * W, C), idx.reshape(B, H * W))
